# Initial kernel scaffold; baseline (speedup 1.0000x reference)
import jax
import jax.numpy as jnp
from jax.experimental import pallas as pl


def kernel(x, edge_index, W1, b1, Wmu, bmu, Wlv, blv):
    raise NotImplementedError("write your pallas kernel here")


# speedup vs baseline: 8.0672x; 8.0672x over previous
import jax
import jax.numpy as jnp
from jax import lax
from jax.experimental import pallas as pl
from jax.experimental.pallas import tpu as pltpu
from jax.experimental.pallas import tpu_sc as plsc

N = 10000
D_IN = 128
D_HID = 128
D_OUT = 64
E = 320000

NC = 2
NS = 16
CH = 125
EPS = E // NS
CPS = EPS // CH
RNG = 2560
ACC_R = RNG + 256
ZPT = ACC_R // NS
ZR = 88
WPT = RNG // NS

_SC_MESH = plsc.VectorSubcoreMesh(
    core_axis_name="c", subcore_axis_name="s", num_cores=NC, num_subcores=NS
)


def _sc_degree_body(dst_hbm, deg_hbm, dstbuf, ones_buf, zdeg, deg_acc):
    c = lax.axis_index("c")
    s = lax.axis_index("s")
    w = c * NS + s

    def fill_ones(k, carry):
        ones_buf[pl.ds(k * 16, 16)] = jnp.ones((16,), jnp.float32)
        return carry

    lax.fori_loop(0, 8, fill_ones, 0)

    @pl.when(s == 0)
    def _zero_acc():
        def zero(i, carry):
            zdeg[pl.ds(i * 16, 16)] = jnp.zeros((16,), jnp.float32)
            return carry

        lax.fori_loop(0, N // 16, zero, 0)
        pltpu.sync_copy(zdeg, deg_acc)

    plsc.subcore_barrier()
    pltpu.sync_copy(dst_hbm.at[pl.ds(w * (CPS // NC), CPS // NC)], dstbuf)

    def chunk(j, carry):
        pltpu.sync_copy(ones_buf.at[pl.ds(0, CH)], deg_acc.at[dstbuf.at[j]], add=True)
        return carry

    lax.fori_loop(0, CPS // NC, chunk, 0)
    plsc.subcore_barrier()

    @pl.when(s == 0)
    def _writeback():
        pltpu.sync_copy(deg_acc, zdeg)
        pltpu.sync_copy(zdeg, deg_hbm.at[pl.ds(c * N, N)])


def _sc_degree(dst2d):
    return pl.kernel(
        _sc_degree_body,
        out_type=jax.ShapeDtypeStruct((NC * N,), jnp.float32),
        mesh=_SC_MESH,
        scratch_types=[
            pltpu.VMEM((CPS // NC, CH), jnp.int32),
            pltpu.VMEM((128,), jnp.float32),
            pltpu.VMEM((N,), jnp.float32),
            pltpu.VMEM_SHARED((N,), jnp.float32),
        ],
    )(dst2d)


def _make_agg_body(base0):
    def _sc_agg_body(src_hbm, dst_hbm, g_hbm, out_hbm,
                     srcbuf, dstbuf, dstloc, rows_a, rows_b, zbuf, acc,
                     gsem_a, gsem_b, ssem_a, ssem_b):
        c = lax.axis_index("c")
        s = lax.axis_index("s")
        base = base0 + c * RNG
        trash = RNG + 16 * s

        def zero_row(r, carry):
            for k in range(D_HID // 16):
                zbuf[r, pl.ds(k * 16, 16)] = jnp.zeros((16,), jnp.float32)
            return carry

        lax.fori_loop(0, ZR, zero_row, 0)
        for i in range(ZPT // ZR):
            pltpu.sync_copy(zbuf, acc.at[pl.ds(s * ZPT + i * ZR, ZR)])

        pltpu.sync_copy(src_hbm.at[pl.ds(s * CPS, CPS)], srcbuf)
        pltpu.sync_copy(dst_hbm.at[pl.ds(s * CPS, CPS)], dstbuf)

        def remap_row(r, carry):
            for k in range(8):
                off = 109 if k == 7 else k * 16
                v = dstbuf[r, pl.ds(off, 16)]
                t = v - base
                ok = (t >= 0) & (t < RNG)
                dstloc[r, pl.ds(off, 16)] = jnp.where(ok, t, trash)
            return carry

        lax.fori_loop(0, CPS, remap_row, 0)
        plsc.subcore_barrier()

        def edge_pair(j, carry):
            c0 = 2 * j
            c1 = 2 * j + 1
            g0 = pltpu.async_copy(g_hbm.at[srcbuf.at[c0]], rows_a, gsem_a)
            g1 = pltpu.async_copy(g_hbm.at[srcbuf.at[c1]], rows_b, gsem_b)
            g0.wait()
            s0 = pltpu.async_copy(rows_a, acc.at[dstloc.at[c0]], ssem_a, add=True)
            g1.wait()
            s1 = pltpu.async_copy(rows_b, acc.at[dstloc.at[c1]], ssem_b, add=True)
            s0.wait()
            s1.wait()
            return carry

        lax.fori_loop(0, CPS // 2, edge_pair, 0)
        plsc.subcore_barrier()
        pltpu.sync_copy(acc.at[pl.ds(s * WPT, WPT)],
                        out_hbm.at[c].at[pl.ds(s * WPT, WPT)])

    return _sc_agg_body


def _sc_aggregate(src2d, dst2d, g, base0):
    return pl.kernel(
        _make_agg_body(base0),
        out_type=jax.ShapeDtypeStruct((NC, RNG, D_HID), jnp.float32),
        mesh=_SC_MESH,
        scratch_types=[
            pltpu.VMEM((CPS, CH), jnp.int32),
            pltpu.VMEM((CPS, CH), jnp.int32),
            pltpu.VMEM((CPS, CH), jnp.int32),
            pltpu.VMEM((CH, D_HID), jnp.float32),
            pltpu.VMEM((CH, D_HID), jnp.float32),
            pltpu.VMEM((ZR, D_HID), jnp.float32),
            pltpu.VMEM_SHARED((ACC_R, D_HID), jnp.float32),
            pltpu.SemaphoreType.DMA,
            pltpu.SemaphoreType.DMA,
            pltpu.SemaphoreType.DMA,
            pltpu.SemaphoreType.DMA,
        ],
    )(src2d, dst2d, g)


def _aggregate(src2d, dst2d, g):
    parts = [
        _sc_aggregate(src2d, dst2d, g, k * NC * RNG).reshape(NC * RNG, D_HID)
        for k in range(2)
    ]
    return jnp.concatenate(parts, axis=0)


_RB = 2000


def _tc_in_body(d0, d1, x, w, o):
    dinv = lax.rsqrt(d0[...] + d1[...] + 1.0)
    o[...] = dinv * jnp.dot(x[...], w[...], preferred_element_type=jnp.float32)


def _tc_in(x, W1, d0, d1):
    return pl.pallas_call(
        _tc_in_body,
        grid=(N // _RB,),
        in_specs=[
            pl.BlockSpec((_RB, 1), lambda i: (i, 0)),
            pl.BlockSpec((_RB, 1), lambda i: (i, 0)),
            pl.BlockSpec((_RB, D_IN), lambda i: (i, 0)),
            pl.BlockSpec((D_IN, D_HID), lambda i: (0, 0)),
        ],
        out_specs=pl.BlockSpec((_RB, D_HID), lambda i: (i, 0)),
        out_shape=jax.ShapeDtypeStruct((N, D_HID), jnp.float32),
    )(d0, d1, x, W1)


def _tc_mid_body(d0, d1, p, g1, b1, w2, o):
    dinv = lax.rsqrt(d0[...] + d1[...] + 1.0)
    h = jnp.maximum(dinv * (p[...] + g1[...]) + b1[...], 0.0)
    o[...] = dinv * jnp.dot(h, w2[...], preferred_element_type=jnp.float32)


def _tc_mid(p, g1, d0, d1, b1, W2):
    return pl.pallas_call(
        _tc_mid_body,
        grid=(N // _RB,),
        in_specs=[
            pl.BlockSpec((_RB, 1), lambda i: (i, 0)),
            pl.BlockSpec((_RB, 1), lambda i: (i, 0)),
            pl.BlockSpec((_RB, D_HID), lambda i: (i, 0)),
            pl.BlockSpec((_RB, D_HID), lambda i: (i, 0)),
            pl.BlockSpec((1, D_HID), lambda i: (0, 0)),
            pl.BlockSpec((D_HID, D_HID), lambda i: (0, 0)),
        ],
        out_specs=pl.BlockSpec((_RB, D_HID), lambda i: (i, 0)),
        out_shape=jax.ShapeDtypeStruct((N, D_HID), jnp.float32),
    )(d0, d1, p, g1, b1, W2)


def _tc_out_body(d0, d1, q, g2, b2, o):
    dinv = lax.rsqrt(d0[...] + d1[...] + 1.0)
    o[...] = dinv * (q[...] + g2[...]) + b2[...]


def _tc_out(q, g2, d0, d1, b2):
    return pl.pallas_call(
        _tc_out_body,
        grid=(N // _RB,),
        in_specs=[
            pl.BlockSpec((_RB, 1), lambda i: (i, 0)),
            pl.BlockSpec((_RB, 1), lambda i: (i, 0)),
            pl.BlockSpec((_RB, D_HID), lambda i: (i, 0)),
            pl.BlockSpec((_RB, D_HID), lambda i: (i, 0)),
            pl.BlockSpec((1, D_HID), lambda i: (0, 0)),
        ],
        out_specs=pl.BlockSpec((_RB, D_HID), lambda i: (i, 0)),
        out_shape=jax.ShapeDtypeStruct((N, D_HID), jnp.float32),
    )(d0, d1, q, g2, b2)


def kernel(x, edge_index, W1, b1, Wmu, bmu, Wlv, blv):
    src = edge_index[0].astype(jnp.int32).reshape(E // CH, CH)
    dst = edge_index[1].astype(jnp.int32).reshape(E // CH, CH)

    degf = _sc_degree(dst)
    d0 = degf[:N].reshape(N, 1)
    d1 = degf[N:].reshape(N, 1)

    g1 = _tc_in(x, W1, d0, d1)
    p = _aggregate(src, dst, g1)

    W2 = jnp.concatenate([Wmu, Wlv], axis=1)
    b2 = jnp.concatenate([bmu, blv]).reshape(1, D_HID)
    g2 = _tc_mid(p, g1, d0, d1, b1.reshape(1, D_HID), W2)

    q = _aggregate(src, dst, g2)
    out = _tc_out(q, g2, d0, d1, b2)
    return out[:, :D_OUT], out[:, D_OUT:]


# speedup vs baseline: 8.4740x; 1.0504x over previous
import jax
import jax.numpy as jnp
from jax import lax
from jax.experimental import pallas as pl
from jax.experimental.pallas import tpu as pltpu
from jax.experimental.pallas import tpu_sc as plsc

N = 10000
D_IN = 128
D_HID = 128
D_OUT = 64
E = 320000

NC = 2
NS = 16
CH = 125
EPS = E // NS
CPS = EPS // CH
RNG = 2560
ACC_R = RNG + 256
ZPT = ACC_R // NS
ZR = 88
WPT = RNG // NS

_SC_MESH = plsc.VectorSubcoreMesh(
    core_axis_name="c", subcore_axis_name="s", num_cores=NC, num_subcores=NS
)


def _sc_degree_body(dst_hbm, deg_hbm, dstbuf, ones_buf, zdeg, deg_acc):
    c = lax.axis_index("c")
    s = lax.axis_index("s")
    w = c * NS + s

    def fill_ones(k, carry):
        ones_buf[pl.ds(k * 16, 16)] = jnp.ones((16,), jnp.float32)
        return carry

    lax.fori_loop(0, 8, fill_ones, 0)

    @pl.when(s == 0)
    def _zero_acc():
        def zero(i, carry):
            zdeg[pl.ds(i * 16, 16)] = jnp.zeros((16,), jnp.float32)
            return carry

        lax.fori_loop(0, N // 16, zero, 0)
        pltpu.sync_copy(zdeg, deg_acc)

    plsc.subcore_barrier()
    pltpu.sync_copy(dst_hbm.at[pl.ds(w * (CPS // NC), CPS // NC)], dstbuf)

    def chunk(j, carry):
        pltpu.sync_copy(ones_buf.at[pl.ds(0, CH)], deg_acc.at[dstbuf.at[j]], add=True)
        return carry

    lax.fori_loop(0, CPS // NC, chunk, 0)
    plsc.subcore_barrier()

    @pl.when(s == 0)
    def _writeback():
        pltpu.sync_copy(deg_acc, zdeg)
        pltpu.sync_copy(zdeg, deg_hbm.at[pl.ds(c * N, N)])


def _sc_degree(dst2d):
    return pl.kernel(
        _sc_degree_body,
        out_type=jax.ShapeDtypeStruct((NC * N,), jnp.float32),
        mesh=_SC_MESH,
        scratch_types=[
            pltpu.VMEM((CPS // NC, CH), jnp.int32),
            pltpu.VMEM((128,), jnp.float32),
            pltpu.VMEM((N,), jnp.float32),
            pltpu.VMEM_SHARED((N,), jnp.float32),
        ],
    )(dst2d)


def _make_agg_body(base0):
    def _sc_agg_body(src_hbm, dst_hbm, g_hbm, out_hbm,
                     srcbuf, dstbuf, rows_a, rows_b, rows_c, rows_d,
                     acc, gsa, gsb, gsc, gsd, ssa, ssb, ssc, ssd):
        c = lax.axis_index("c")
        s = lax.axis_index("s")
        base = base0 + c * RNG
        trash = RNG + 16 * s

        def zero_row(r, carry):
            for k in range(D_HID // 16):
                rows_a[r, pl.ds(k * 16, 16)] = jnp.zeros((16,), jnp.float32)
            return carry

        lax.fori_loop(0, ZR, zero_row, 0)
        for i in range(ZPT // ZR):
            pltpu.sync_copy(rows_a.at[pl.ds(0, ZR)],
                            acc.at[pl.ds(s * ZPT + i * ZR, ZR)])

        pltpu.sync_copy(src_hbm.at[pl.ds(s * CPS, CPS)], srcbuf)
        pltpu.sync_copy(dst_hbm.at[pl.ds(s * CPS, CPS)], dstbuf)

        lanes = lax.iota(jnp.int32, 16)

        def remap_row(r, carry):
            for k in range(8):
                off = 109 if k == 7 else k * 16
                v = dstbuf[r, pl.ds(off, 16)]
                t = v - base
                ok = (t >= 0) & (t < RNG)
                rem = jnp.where(ok, t, trash)
                if k == 7:
                    rem = jnp.where(lanes >= 3, rem, v)
                dstbuf[r, pl.ds(off, 16)] = rem
            return carry

        lax.fori_loop(0, CPS, remap_row, 0)
        plsc.subcore_barrier()

        def gather(j, buf, sem):
            return pltpu.async_copy(g_hbm.at[srcbuf.at[j]], buf, sem)

        def scatter(j, buf, sem):
            return pltpu.async_copy(buf, acc.at[dstbuf.at[j]], sem, add=True)

        def wait_gather(buf, sem):
            pltpu.make_async_copy(g_hbm.at[srcbuf.at[0]], buf, sem).wait()

        def wait_scatter(buf, sem):
            pltpu.make_async_copy(buf, acc.at[dstbuf.at[0]], sem).wait()

        last = CPS - 1

        gather(0, rows_a, gsa)
        gather(1, rows_b, gsb)
        wait_gather(rows_a, gsa)
        scatter(0, rows_a, ssa)
        wait_gather(rows_b, gsb)
        scatter(1, rows_b, ssb)
        gather(2, rows_c, gsc)
        gather(3, rows_d, gsd)
        wait_gather(rows_c, gsc)
        scatter(2, rows_c, ssc)
        wait_gather(rows_d, gsd)
        scatter(3, rows_d, ssd)
        wait_scatter(rows_a, ssa)
        gather(4, rows_a, gsa)
        wait_scatter(rows_b, ssb)
        gather(5, rows_b, gsb)

        def pipe(j2, carry):
            c0 = 4 * j2
            wait_gather(rows_a, gsa)
            scatter(c0, rows_a, ssa)
            wait_gather(rows_b, gsb)
            scatter(c0 + 1, rows_b, ssb)
            wait_scatter(rows_c, ssc)
            gather(c0 + 2, rows_c, gsc)
            wait_scatter(rows_d, ssd)
            gather(c0 + 3, rows_d, gsd)
            wait_gather(rows_c, gsc)
            scatter(c0 + 2, rows_c, ssc)
            wait_gather(rows_d, gsd)
            scatter(c0 + 3, rows_d, ssd)
            wait_scatter(rows_a, ssa)
            gather(jnp.minimum(c0 + 4, last), rows_a, gsa)
            wait_scatter(rows_b, ssb)
            gather(jnp.minimum(c0 + 5, last), rows_b, gsb)
            return carry

        lax.fori_loop(1, CPS // 4, pipe, 0)
        wait_gather(rows_a, gsa)
        wait_gather(rows_b, gsb)
        wait_scatter(rows_c, ssc)
        wait_scatter(rows_d, ssd)

        plsc.subcore_barrier()
        pltpu.sync_copy(acc.at[pl.ds(s * WPT, WPT)],
                        out_hbm.at[c].at[pl.ds(s * WPT, WPT)])

    return _sc_agg_body


def _sc_aggregate(src2d, dst2d, g, base0):
    return pl.kernel(
        _make_agg_body(base0),
        out_type=jax.ShapeDtypeStruct((NC, RNG, D_HID), jnp.float32),
        mesh=_SC_MESH,
        scratch_types=[
            pltpu.VMEM((CPS, CH), jnp.int32),
            pltpu.VMEM((CPS, CH), jnp.int32),
            pltpu.VMEM((CH, D_HID), jnp.float32),
            pltpu.VMEM((CH, D_HID), jnp.float32),
            pltpu.VMEM((CH, D_HID), jnp.float32),
            pltpu.VMEM((CH, D_HID), jnp.float32),
            pltpu.VMEM_SHARED((ACC_R, D_HID), jnp.float32),
            pltpu.SemaphoreType.DMA,
            pltpu.SemaphoreType.DMA,
            pltpu.SemaphoreType.DMA,
            pltpu.SemaphoreType.DMA,
            pltpu.SemaphoreType.DMA,
            pltpu.SemaphoreType.DMA,
            pltpu.SemaphoreType.DMA,
            pltpu.SemaphoreType.DMA,
        ],
    )(src2d, dst2d, g)


def _aggregate(src2d, dst2d, g):
    parts = [
        _sc_aggregate(src2d, dst2d, g, k * NC * RNG).reshape(NC * RNG, D_HID)
        for k in range(2)
    ]
    return jnp.concatenate(parts, axis=0)


_RB = 2000


def _tc_in_body(d0, d1, x, w, o):
    dinv = lax.rsqrt(d0[...] + d1[...] + 1.0)
    o[...] = dinv * jnp.dot(x[...], w[...], preferred_element_type=jnp.float32)


def _tc_in(x, W1, d0, d1):
    return pl.pallas_call(
        _tc_in_body,
        grid=(N // _RB,),
        in_specs=[
            pl.BlockSpec((_RB, 1), lambda i: (i, 0)),
            pl.BlockSpec((_RB, 1), lambda i: (i, 0)),
            pl.BlockSpec((_RB, D_IN), lambda i: (i, 0)),
            pl.BlockSpec((D_IN, D_HID), lambda i: (0, 0)),
        ],
        out_specs=pl.BlockSpec((_RB, D_HID), lambda i: (i, 0)),
        out_shape=jax.ShapeDtypeStruct((N, D_HID), jnp.float32),
    )(d0, d1, x, W1)


def _tc_mid_body(d0, d1, p, g1, b1, w2, o):
    dinv = lax.rsqrt(d0[...] + d1[...] + 1.0)
    h = jnp.maximum(dinv * (p[...] + g1[...]) + b1[...], 0.0)
    o[...] = dinv * jnp.dot(h, w2[...], preferred_element_type=jnp.float32)


def _tc_mid(p, g1, d0, d1, b1, W2):
    return pl.pallas_call(
        _tc_mid_body,
        grid=(N // _RB,),
        in_specs=[
            pl.BlockSpec((_RB, 1), lambda i: (i, 0)),
            pl.BlockSpec((_RB, 1), lambda i: (i, 0)),
            pl.BlockSpec((_RB, D_HID), lambda i: (i, 0)),
            pl.BlockSpec((_RB, D_HID), lambda i: (i, 0)),
            pl.BlockSpec((1, D_HID), lambda i: (0, 0)),
            pl.BlockSpec((D_HID, D_HID), lambda i: (0, 0)),
        ],
        out_specs=pl.BlockSpec((_RB, D_HID), lambda i: (i, 0)),
        out_shape=jax.ShapeDtypeStruct((N, D_HID), jnp.float32),
    )(d0, d1, p, g1, b1, W2)


def _tc_out_body(d0, d1, q, g2, b2, o):
    dinv = lax.rsqrt(d0[...] + d1[...] + 1.0)
    o[...] = dinv * (q[...] + g2[...]) + b2[...]


def _tc_out(q, g2, d0, d1, b2):
    return pl.pallas_call(
        _tc_out_body,
        grid=(N // _RB,),
        in_specs=[
            pl.BlockSpec((_RB, 1), lambda i: (i, 0)),
            pl.BlockSpec((_RB, 1), lambda i: (i, 0)),
            pl.BlockSpec((_RB, D_HID), lambda i: (i, 0)),
            pl.BlockSpec((_RB, D_HID), lambda i: (i, 0)),
            pl.BlockSpec((1, D_HID), lambda i: (0, 0)),
        ],
        out_specs=pl.BlockSpec((_RB, D_HID), lambda i: (i, 0)),
        out_shape=jax.ShapeDtypeStruct((N, D_HID), jnp.float32),
    )(d0, d1, q, g2, b2)


def kernel(x, edge_index, W1, b1, Wmu, bmu, Wlv, blv):
    src = edge_index[0].astype(jnp.int32).reshape(E // CH, CH)
    dst = edge_index[1].astype(jnp.int32).reshape(E // CH, CH)

    degf = _sc_degree(dst)
    d0 = degf[:N].reshape(N, 1)
    d1 = degf[N:].reshape(N, 1)

    g1 = _tc_in(x, W1, d0, d1)
    p = _aggregate(src, dst, g1)

    W2 = jnp.concatenate([Wmu, Wlv], axis=1)
    b2 = jnp.concatenate([bmu, blv]).reshape(1, D_HID)
    g2 = _tc_mid(p, g1, d0, d1, b1.reshape(1, D_HID), W2)

    q = _aggregate(src, dst, g2)
    out = _tc_out(q, g2, d0, d1, b2)
    return out[:, :D_OUT], out[:, D_OUT:]


# speedup vs baseline: 13.1969x; 1.5573x over previous
import jax
import jax.numpy as jnp
from jax import lax
from jax.experimental import pallas as pl
from jax.experimental.pallas import tpu as pltpu
from jax.experimental.pallas import tpu_sc as plsc

N = 10000
D_IN = 128
D_HID = 128
D_OUT = 64
E = 320000

NC = 2
NS = 16
CH = 125
EPS = E // NS
CPS = EPS // CH
RNG = 2560
ACC_R = RNG + 256
ZPT = ACC_R // NS
ZR = 88
WPT = RNG // NS

_SC_MESH = plsc.VectorSubcoreMesh(
    core_axis_name="c", subcore_axis_name="s", num_cores=NC, num_subcores=NS
)


def _sc_degree_body(dst_hbm, deg_hbm, dstbuf, ones_buf, zdeg, deg_acc):
    c = lax.axis_index("c")
    s = lax.axis_index("s")
    w = c * NS + s

    def fill_ones(k, carry):
        ones_buf[pl.ds(k * 16, 16)] = jnp.ones((16,), jnp.float32)
        return carry

    lax.fori_loop(0, 8, fill_ones, 0)

    @pl.when(s == 0)
    def _zero_acc():
        def zero(i, carry):
            zdeg[pl.ds(i * 16, 16)] = jnp.zeros((16,), jnp.float32)
            return carry

        lax.fori_loop(0, N // 16, zero, 0)
        pltpu.sync_copy(zdeg, deg_acc)

    plsc.subcore_barrier()
    pltpu.sync_copy(dst_hbm.at[pl.ds(w * (CPS // NC), CPS // NC)], dstbuf)

    def chunk(j, carry):
        pltpu.sync_copy(ones_buf.at[pl.ds(0, CH)], deg_acc.at[dstbuf.at[j]], add=True)
        return carry

    lax.fori_loop(0, CPS // NC, chunk, 0)
    plsc.subcore_barrier()

    @pl.when(s == 0)
    def _writeback():
        pltpu.sync_copy(deg_acc, zdeg)
        pltpu.sync_copy(zdeg, deg_hbm.at[pl.ds(c * N, N)])


def _sc_degree(dst2d):
    return pl.kernel(
        _sc_degree_body,
        out_type=jax.ShapeDtypeStruct((NC * N,), jnp.float32),
        mesh=_SC_MESH,
        scratch_types=[
            pltpu.VMEM((CPS // NC, CH), jnp.int32),
            pltpu.VMEM((128,), jnp.float32),
            pltpu.VMEM((N,), jnp.float32),
            pltpu.VMEM_SHARED((N,), jnp.float32),
        ],
    )(dst2d)


def _make_agg_body(base0):
    def _sc_agg_body(src_hbm, dst_hbm, g_hbm, out_hbm,
                     srcbuf, dstbuf, rows_a, rows_b, rows_c, rows_d,
                     acc, gsa, gsb, gsc, gsd, ssa, ssb, ssc, ssd):
        c = lax.axis_index("c")
        s = lax.axis_index("s")
        base = base0 + c * RNG
        trash = RNG + 16 * s + lax.iota(jnp.int32, 16)

        def zero_row(r, carry):
            for k in range(D_HID // 16):
                rows_a[r, pl.ds(k * 16, 16)] = jnp.zeros((16,), jnp.float32)
            return carry

        lax.fori_loop(0, ZR, zero_row, 0)
        for i in range(ZPT // ZR):
            pltpu.sync_copy(rows_a.at[pl.ds(0, ZR)],
                            acc.at[pl.ds(s * ZPT + i * ZR, ZR)])

        pltpu.sync_copy(src_hbm.at[pl.ds(s * CPS, CPS)], srcbuf)
        pltpu.sync_copy(dst_hbm.at[pl.ds(s * CPS, CPS)], dstbuf)

        lanes = lax.iota(jnp.int32, 16)

        def remap_row(r, carry):
            for k in range(8):
                off = 109 if k == 7 else k * 16
                v = dstbuf[r, pl.ds(off, 16)]
                t = v - base
                ok = (t >= 0) & (t < RNG)
                rem = jnp.where(ok, t, trash)
                if k == 7:
                    rem = jnp.where(lanes >= 3, rem, v)
                dstbuf[r, pl.ds(off, 16)] = rem
            return carry

        lax.fori_loop(0, CPS, remap_row, 0)
        plsc.subcore_barrier()

        def gather(j, buf, sem):
            return pltpu.async_copy(g_hbm.at[srcbuf.at[j]], buf, sem)

        def scatter(j, buf, sem):
            return pltpu.async_copy(buf, acc.at[dstbuf.at[j]], sem, add=True)

        def wait_gather(buf, sem):
            pltpu.make_async_copy(g_hbm.at[srcbuf.at[0]], buf, sem).wait()

        def wait_scatter(buf, sem):
            pltpu.make_async_copy(buf, acc.at[dstbuf.at[0]], sem).wait()

        last = CPS - 1

        gather(0, rows_a, gsa)
        gather(1, rows_b, gsb)
        wait_gather(rows_a, gsa)
        scatter(0, rows_a, ssa)
        wait_gather(rows_b, gsb)
        scatter(1, rows_b, ssb)
        gather(2, rows_c, gsc)
        gather(3, rows_d, gsd)
        wait_gather(rows_c, gsc)
        scatter(2, rows_c, ssc)
        wait_gather(rows_d, gsd)
        scatter(3, rows_d, ssd)
        wait_scatter(rows_a, ssa)
        gather(4, rows_a, gsa)
        wait_scatter(rows_b, ssb)
        gather(5, rows_b, gsb)

        def pipe(j2, carry):
            c0 = 4 * j2
            wait_gather(rows_a, gsa)
            scatter(c0, rows_a, ssa)
            wait_gather(rows_b, gsb)
            scatter(c0 + 1, rows_b, ssb)
            wait_scatter(rows_c, ssc)
            gather(c0 + 2, rows_c, gsc)
            wait_scatter(rows_d, ssd)
            gather(c0 + 3, rows_d, gsd)
            wait_gather(rows_c, gsc)
            scatter(c0 + 2, rows_c, ssc)
            wait_gather(rows_d, gsd)
            scatter(c0 + 3, rows_d, ssd)
            wait_scatter(rows_a, ssa)
            gather(jnp.minimum(c0 + 4, last), rows_a, gsa)
            wait_scatter(rows_b, ssb)
            gather(jnp.minimum(c0 + 5, last), rows_b, gsb)
            return carry

        lax.fori_loop(1, CPS // 4, pipe, 0)
        wait_gather(rows_a, gsa)
        wait_gather(rows_b, gsb)
        wait_scatter(rows_c, ssc)
        wait_scatter(rows_d, ssd)

        plsc.subcore_barrier()
        pltpu.sync_copy(acc.at[pl.ds(s * WPT, WPT)],
                        out_hbm.at[c].at[pl.ds(s * WPT, WPT)])

    return _sc_agg_body


def _sc_aggregate(src2d, dst2d, g, base0):
    return pl.kernel(
        _make_agg_body(base0),
        out_type=jax.ShapeDtypeStruct((NC, RNG, D_HID), jnp.float32),
        mesh=_SC_MESH,
        scratch_types=[
            pltpu.VMEM((CPS, CH), jnp.int32),
            pltpu.VMEM((CPS, CH), jnp.int32),
            pltpu.VMEM((CH, D_HID), jnp.float32),
            pltpu.VMEM((CH, D_HID), jnp.float32),
            pltpu.VMEM((CH, D_HID), jnp.float32),
            pltpu.VMEM((CH, D_HID), jnp.float32),
            pltpu.VMEM_SHARED((ACC_R, D_HID), jnp.float32),
            pltpu.SemaphoreType.DMA,
            pltpu.SemaphoreType.DMA,
            pltpu.SemaphoreType.DMA,
            pltpu.SemaphoreType.DMA,
            pltpu.SemaphoreType.DMA,
            pltpu.SemaphoreType.DMA,
            pltpu.SemaphoreType.DMA,
            pltpu.SemaphoreType.DMA,
        ],
    )(src2d, dst2d, g)


def _aggregate(src2d, dst2d, g):
    parts = [
        _sc_aggregate(src2d, dst2d, g, k * NC * RNG).reshape(NC * RNG, D_HID)
        for k in range(2)
    ]
    return jnp.concatenate(parts, axis=0)


_RB = 2000


def _tc_in_body(d0, d1, x, w, o):
    dinv = lax.rsqrt(d0[...] + d1[...] + 1.0)
    o[...] = dinv * jnp.dot(x[...], w[...], preferred_element_type=jnp.float32)


def _tc_in(x, W1, d0, d1):
    return pl.pallas_call(
        _tc_in_body,
        grid=(N // _RB,),
        in_specs=[
            pl.BlockSpec((_RB, 1), lambda i: (i, 0)),
            pl.BlockSpec((_RB, 1), lambda i: (i, 0)),
            pl.BlockSpec((_RB, D_IN), lambda i: (i, 0)),
            pl.BlockSpec((D_IN, D_HID), lambda i: (0, 0)),
        ],
        out_specs=pl.BlockSpec((_RB, D_HID), lambda i: (i, 0)),
        out_shape=jax.ShapeDtypeStruct((N, D_HID), jnp.float32),
    )(d0, d1, x, W1)


def _tc_mid_body(d0, d1, p, g1, b1, w2, o):
    dinv = lax.rsqrt(d0[...] + d1[...] + 1.0)
    h = jnp.maximum(dinv * (p[...] + g1[...]) + b1[...], 0.0)
    o[...] = dinv * jnp.dot(h, w2[...], preferred_element_type=jnp.float32)


def _tc_mid(p, g1, d0, d1, b1, W2):
    return pl.pallas_call(
        _tc_mid_body,
        grid=(N // _RB,),
        in_specs=[
            pl.BlockSpec((_RB, 1), lambda i: (i, 0)),
            pl.BlockSpec((_RB, 1), lambda i: (i, 0)),
            pl.BlockSpec((_RB, D_HID), lambda i: (i, 0)),
            pl.BlockSpec((_RB, D_HID), lambda i: (i, 0)),
            pl.BlockSpec((1, D_HID), lambda i: (0, 0)),
            pl.BlockSpec((D_HID, D_HID), lambda i: (0, 0)),
        ],
        out_specs=pl.BlockSpec((_RB, D_HID), lambda i: (i, 0)),
        out_shape=jax.ShapeDtypeStruct((N, D_HID), jnp.float32),
    )(d0, d1, p, g1, b1, W2)


def _tc_out_body(d0, d1, q, g2, b2, o):
    dinv = lax.rsqrt(d0[...] + d1[...] + 1.0)
    o[...] = dinv * (q[...] + g2[...]) + b2[...]


def _tc_out(q, g2, d0, d1, b2):
    return pl.pallas_call(
        _tc_out_body,
        grid=(N // _RB,),
        in_specs=[
            pl.BlockSpec((_RB, 1), lambda i: (i, 0)),
            pl.BlockSpec((_RB, 1), lambda i: (i, 0)),
            pl.BlockSpec((_RB, D_HID), lambda i: (i, 0)),
            pl.BlockSpec((_RB, D_HID), lambda i: (i, 0)),
            pl.BlockSpec((1, D_HID), lambda i: (0, 0)),
        ],
        out_specs=pl.BlockSpec((_RB, D_HID), lambda i: (i, 0)),
        out_shape=jax.ShapeDtypeStruct((N, D_HID), jnp.float32),
    )(d0, d1, q, g2, b2)


def kernel(x, edge_index, W1, b1, Wmu, bmu, Wlv, blv):
    src = edge_index[0].astype(jnp.int32).reshape(E // CH, CH)
    dst = edge_index[1].astype(jnp.int32).reshape(E // CH, CH)

    degf = _sc_degree(dst)
    d0 = degf[:N].reshape(N, 1)
    d1 = degf[N:].reshape(N, 1)

    g1 = _tc_in(x, W1, d0, d1)
    p = _aggregate(src, dst, g1)

    W2 = jnp.concatenate([Wmu, Wlv], axis=1)
    b2 = jnp.concatenate([bmu, blv]).reshape(1, D_HID)
    g2 = _tc_mid(p, g1, d0, d1, b1.reshape(1, D_HID), W2)

    q = _aggregate(src, dst, g2)
    out = _tc_out(q, g2, d0, d1, b2)
    return out[:, :D_OUT], out[:, D_OUT:]
